# baseline (speedup 1.0000x reference)
import dataclasses

import jax
import jax.numpy as jnp
from jax import lax
from jax.experimental import pallas as pl
from jax.experimental.pallas import tpu as pltpu
from jax.experimental.pallas import tpu_sc as plsc

B_BOX = 2
C_CLS = 20
LAMBDA_COORD = 5.0
LAMBDA_NOOBJ = 0.5
N_ELEM = B_BOX * 5 + C_CLS
BATCH = 4096
S = 7
FEAT = S * S * N_ELEM

SC_ROWS = 2048
TC_ROWS = BATCH - SC_ROWS

NC = 2
NS = 16
NW = NC * NS
L = 16
ROWS_PER_W = SC_ROWS // NW
CHUNK = 16
N_CHUNKS = ROWS_PER_W // CHUNK

PACK = 4
W = PACK * N_ELEM
WP = 128
N4 = TC_ROWS * S * S // PACK
TC_GRID = 28
TC_BLK = N4 // TC_GRID


def _splat(v):
    return jnp.full((L,), v, jnp.int32)


def _box_loss(pg, tg):
    d = [pg[k] - tg[k] for k in range(10)]
    s0 = d[0] * d[0] + d[1] * d[1] + d[2] * d[2] + d[3] * d[3]
    s1 = d[5] * d[5] + d[6] * d[6] + d[7] * d[7] + d[8] * d[8]
    c0 = d[4] * d[4]
    c1 = d[9] * d[9]

    def inter(px1, py1, px2, py2, tx1, ty1, tx2, ty2):
        w = jnp.maximum(jnp.minimum(px2, tx2) - jnp.maximum(px1, tx1), 0.0)
        h = jnp.maximum(jnp.minimum(py2, ty2) - jnp.maximum(py1, ty1), 0.0)
        return w * h

    ap0 = (pg[2] - pg[0]) * (pg[3] - pg[1])
    ap1 = (pg[7] - pg[5]) * (pg[8] - pg[6])
    at0 = (tg[2] - tg[0]) * (tg[3] - tg[1])
    at1 = (tg[7] - tg[5]) * (tg[8] - tg[6])
    i00 = inter(pg[0], pg[1], pg[2], pg[3], tg[0], tg[1], tg[2], tg[3])
    i10 = inter(pg[5], pg[6], pg[7], pg[8], tg[0], tg[1], tg[2], tg[3])
    i01 = inter(pg[0], pg[1], pg[2], pg[3], tg[5], tg[6], tg[7], tg[8])
    i11 = inter(pg[5], pg[6], pg[7], pg[8], tg[5], tg[6], tg[7], tg[8])
    d00 = ap0 + at0 - i00
    d10 = ap1 + at0 - i10
    d01 = ap0 + at1 - i01
    d11 = ap1 + at1 - i11

    dl0 = i10 * d00 - i00 * d10
    dl1 = i11 * d01 - i01 * d11
    m0 = jnp.where(d10 * d00 < 0.0, -dl0, dl0) > 0.0
    m1 = jnp.where(d11 * d01 < 0.0, -dl1, dl1) > 0.0

    coordf = jnp.where(tg[5] > 0.0, 1.0, 0.0)
    nw = jnp.where(tg[5] == 0.0, LAMBDA_NOOBJ, 0.0)
    r0 = jnp.where(~(m0 & m1), coordf, 0.0)
    r1 = jnp.where(m0 | m1, coordf, 0.0)

    return (
        LAMBDA_COORD * (r0 * s0 + r1 * s1)
        + r0 * c0 + r1 * c1
        + nw * (c0 + c1)
    )


def _compiler_params_sc():
    cp = pltpu.CompilerParams()
    if "needs_layout_passes" in pltpu.CompilerParams.__dataclass_fields__:
        cp = dataclasses.replace(cp, needs_layout_passes=False)
    return cp


def _sc_partials(pred2, targ2):
    mesh = plsc.VectorSubcoreMesh(core_axis_name="c", subcore_axis_name="s")

    @pl.kernel(
        out_type=jax.ShapeDtypeStruct((NW, L), jnp.float32),
        mesh=mesh,
        compiler_params=_compiler_params_sc(),
        scratch_types=[
            pltpu.VMEM((CHUNK, FEAT), jnp.float32),
            pltpu.VMEM((CHUNK, FEAT), jnp.float32),
            pltpu.VMEM((CHUNK, FEAT), jnp.float32),
            pltpu.VMEM((CHUNK, FEAT), jnp.float32),
            pltpu.VMEM((1, L), jnp.float32),
            pltpu.SemaphoreType.DMA,
            pltpu.SemaphoreType.DMA,
        ],
    )
    def sc_kernel(p_hbm, t_hbm, o_hbm, p_v0, t_v0, p_v1, t_v1, acc_v,
                  sem0, sem1):
        wid = lax.axis_index("s") * NC + lax.axis_index("c")
        base = wid * ROWS_PER_W
        acc_v[0] = jnp.zeros((L,), jnp.float32)
        row_iota = lax.iota(jnp.int32, L)
        last = base + (N_CHUNKS - 1) * CHUNK

        def start(ch, p_v, t_v, sem):
            r0_ = jnp.minimum(base + ch * CHUNK, last)
            c1 = pltpu.async_copy(p_hbm.at[pl.ds(r0_, CHUNK)], p_v, sem)
            c2 = pltpu.async_copy(t_hbm.at[pl.ds(r0_, CHUNK)], t_v, sem)
            return c1, c2

        def compute(p_v, t_v):
            @pl.loop(0, S)
            def _(j1):
                @pl.loop(0, S)
                def _(j2):
                    cbase = (j1 * S + j2) * N_ELEM
                    pg = [
                        plsc.load_gather(p_v, [row_iota, _splat(cbase + k)])
                        for k in range(10)
                    ]
                    tg = [
                        plsc.load_gather(t_v, [row_iota, _splat(cbase + k)])
                        for k in range(10)
                    ]
                    box = _box_loss(pg, tg)
                    coordf = jnp.where(tg[5] > 0.0, 1.0, 0.0)
                    cls = None
                    for k in range(10, 30):
                        pk = plsc.load_gather(p_v, [row_iota, _splat(cbase + k)])
                        tk = plsc.load_gather(t_v, [row_iota, _splat(cbase + k)])
                        dd = pk - tk
                        dd = dd * dd
                        cls = dd if cls is None else cls + dd
                    acc_v[0] = acc_v[0] + box + coordf * cls

        a1, a2 = start(0, p_v0, t_v0, sem0)

        @pl.loop(0, N_CHUNKS // 2)
        def _(ch2):
            b1, b2 = start(2 * ch2 + 1, p_v1, t_v1, sem1)
            pltpu.make_async_copy(p_hbm.at[pl.ds(base, CHUNK)], p_v0, sem0).wait()
            pltpu.make_async_copy(t_hbm.at[pl.ds(base, CHUNK)], t_v0, sem0).wait()
            compute(p_v0, t_v0)
            start(2 * ch2 + 2, p_v0, t_v0, sem0)
            pltpu.make_async_copy(p_hbm.at[pl.ds(base, CHUNK)], p_v1, sem1).wait()
            pltpu.make_async_copy(t_hbm.at[pl.ds(base, CHUNK)], t_v1, sem1).wait()
            compute(p_v1, t_v1)

        pltpu.make_async_copy(p_hbm.at[pl.ds(base, CHUNK)], p_v0, sem0).wait()
        pltpu.make_async_copy(t_hbm.at[pl.ds(base, CHUNK)], t_v0, sem0).wait()

        pltpu.sync_copy(acc_v, o_hbm.at[pl.ds(wid, 1)])

    return sc_kernel(pred2, targ2)


def _rl(x, k):
    return pltpu.roll(x, WP - k, 1)


def _tc_block_body(p_ref, t_ref, o_ref):
    zpad = jnp.zeros((TC_BLK, WP - W), jnp.float32)
    x = jnp.concatenate([p_ref[...], zpad], axis=1)
    y = jnp.concatenate([t_ref[...], zpad], axis=1)

    d = x - y
    d2 = d * d
    s1 = d2 + _rl(d2, 1)
    s2 = s1 + _rl(s1, 2)
    s4 = s2 + _rl(s2, 4)
    s8 = s4 + _rl(s4, 8)
    s_box0 = s2
    s_box1 = _rl(s2, 5)
    s_class = _rl(s8, 10) + _rl(s2, 26)
    c0 = _rl(d2, 4)
    c1 = _rl(d2, 9)

    x2, x5, x7 = _rl(x, 2), _rl(x, 5), _rl(x, 7)
    y2, y5, y7 = _rl(y, 2), _rl(y, 5), _rl(y, 7)

    def wh(ahi, bhi, alo, blo):
        return jnp.maximum(jnp.minimum(ahi, bhi) - jnp.maximum(alo, blo), 0.0)

    wh_c = wh(x2, y2, x, y)
    wh_a = wh(x7, y2, x5, y)
    wh_b = wh(x2, y7, x, y5)
    i_c = wh_c * _rl(wh_c, 1)
    i_a = wh_a * _rl(wh_a, 1)
    i_b = wh_b * _rl(wh_b, 1)

    ex = x2 - x
    ap = ex * _rl(ex, 1)
    ey = y2 - y
    at = ey * _rl(ey, 1)
    ap5 = _rl(ap, 5)
    at5 = _rl(at, 5)

    d_c = ap + at - i_c
    d_a = ap5 + at - i_a
    d_b = ap + at5 - i_b
    i_c5 = _rl(i_c, 5)
    d_c5 = _rl(d_c, 5)

    dl0 = i_a * d_c - i_c * d_a
    dl1 = i_c5 * d_b - i_b * d_c5
    m0 = jnp.where(d_a * d_c < 0.0, -dl0, dl0) > 0.0
    m1 = jnp.where(d_c5 * d_b < 0.0, -dl1, dl1) > 0.0

    coordf = jnp.where(y5 > 0.0, 1.0, 0.0)
    nw = jnp.where(y5 == 0.0, LAMBDA_NOOBJ, 0.0)
    r0 = jnp.where(~(m0 & m1), coordf, 0.0)
    r1 = jnp.where(m0 | m1, coordf, 0.0)

    per_cell = (
        LAMBDA_COORD * (r0 * s_box0 + r1 * s_box1)
        + r0 * c0 + r1 * c1
        + nw * (c0 + c1)
        + coordf * s_class
    )
    lane = jax.lax.broadcasted_iota(jnp.int32, (TC_BLK, WP), 1)
    masked = jnp.where((lane % N_ELEM == 0) & (lane < W), per_cell, 0.0)
    o_ref[...] = jnp.sum(masked).reshape(1, 1, 1)


def _tc_partials(p4, t4):
    return pl.pallas_call(
        _tc_block_body,
        grid=(TC_GRID,),
        in_specs=[
            pl.BlockSpec((TC_BLK, W), lambda i: (i, 0)),
            pl.BlockSpec((TC_BLK, W), lambda i: (i, 0)),
        ],
        out_specs=pl.BlockSpec((1, 1, 1), lambda i: (i, 0, 0)),
        out_shape=jax.ShapeDtypeStruct((TC_GRID, 1, 1), jnp.float32),
    )(p4, t4)


def kernel(pred_tensor, target_tensor):
    targ2 = target_tensor.reshape(BATCH, FEAT)
    tc_part = _tc_partials(
        pred_tensor[SC_ROWS:].reshape(N4, W),
        targ2[SC_ROWS:].reshape(N4, W),
    )
    sc_part = _sc_partials(pred_tensor[:SC_ROWS], targ2[:SC_ROWS])
    return jnp.sum(sc_part) + jnp.sum(tc_part)


# speedup vs baseline: 1.0272x; 1.0272x over previous
import dataclasses

import jax
import jax.numpy as jnp
from jax import lax
from jax.experimental import pallas as pl
from jax.experimental.pallas import tpu as pltpu
from jax.experimental.pallas import tpu_sc as plsc

B_BOX = 2
C_CLS = 20
LAMBDA_COORD = 5.0
LAMBDA_NOOBJ = 0.5
N_ELEM = B_BOX * 5 + C_CLS
BATCH = 4096
S = 7
FEAT = S * S * N_ELEM

SC_ROWS = 2560
TC_ROWS = BATCH - SC_ROWS

NC = 2
NS = 16
NW = NC * NS
L = 16
ROWS_PER_W = SC_ROWS // NW
CHUNK = 16
N_CHUNKS = ROWS_PER_W // CHUNK

PACK = 4
W = PACK * N_ELEM
WP = 128
N4 = TC_ROWS * S * S // PACK
TC_GRID = 28
TC_BLK = N4 // TC_GRID


def _splat(v):
    return jnp.full((L,), v, jnp.int32)


def _box_loss(pg, tg):
    d = [pg[k] - tg[k] for k in range(10)]
    s0 = d[0] * d[0] + d[1] * d[1] + d[2] * d[2] + d[3] * d[3]
    s1 = d[5] * d[5] + d[6] * d[6] + d[7] * d[7] + d[8] * d[8]
    c0 = d[4] * d[4]
    c1 = d[9] * d[9]

    def inter(px1, py1, px2, py2, tx1, ty1, tx2, ty2):
        w = jnp.maximum(jnp.minimum(px2, tx2) - jnp.maximum(px1, tx1), 0.0)
        h = jnp.maximum(jnp.minimum(py2, ty2) - jnp.maximum(py1, ty1), 0.0)
        return w * h

    ap0 = (pg[2] - pg[0]) * (pg[3] - pg[1])
    ap1 = (pg[7] - pg[5]) * (pg[8] - pg[6])
    at0 = (tg[2] - tg[0]) * (tg[3] - tg[1])
    at1 = (tg[7] - tg[5]) * (tg[8] - tg[6])
    i00 = inter(pg[0], pg[1], pg[2], pg[3], tg[0], tg[1], tg[2], tg[3])
    i10 = inter(pg[5], pg[6], pg[7], pg[8], tg[0], tg[1], tg[2], tg[3])
    i01 = inter(pg[0], pg[1], pg[2], pg[3], tg[5], tg[6], tg[7], tg[8])
    i11 = inter(pg[5], pg[6], pg[7], pg[8], tg[5], tg[6], tg[7], tg[8])
    d00 = ap0 + at0 - i00
    d10 = ap1 + at0 - i10
    d01 = ap0 + at1 - i01
    d11 = ap1 + at1 - i11

    dl0 = i10 * d00 - i00 * d10
    dl1 = i11 * d01 - i01 * d11
    m0 = jnp.where(d10 * d00 < 0.0, -dl0, dl0) > 0.0
    m1 = jnp.where(d11 * d01 < 0.0, -dl1, dl1) > 0.0

    coordf = jnp.where(tg[5] > 0.0, 1.0, 0.0)
    nw = jnp.where(tg[5] == 0.0, LAMBDA_NOOBJ, 0.0)
    r0 = jnp.where(~(m0 & m1), coordf, 0.0)
    r1 = jnp.where(m0 | m1, coordf, 0.0)

    return (
        LAMBDA_COORD * (r0 * s0 + r1 * s1)
        + r0 * c0 + r1 * c1
        + nw * (c0 + c1)
    )


def _compiler_params_sc():
    cp = pltpu.CompilerParams()
    if "needs_layout_passes" in pltpu.CompilerParams.__dataclass_fields__:
        cp = dataclasses.replace(cp, needs_layout_passes=False)
    return cp


def _sc_partials(pred2, targ2):
    mesh = plsc.VectorSubcoreMesh(core_axis_name="c", subcore_axis_name="s")

    @pl.kernel(
        out_type=jax.ShapeDtypeStruct((NW, L), jnp.float32),
        mesh=mesh,
        compiler_params=_compiler_params_sc(),
        scratch_types=[
            pltpu.VMEM((CHUNK, FEAT), jnp.float32),
            pltpu.VMEM((CHUNK, FEAT), jnp.float32),
            pltpu.VMEM((1, L), jnp.float32),
            pltpu.SemaphoreType.DMA,
        ],
    )
    def sc_kernel(p_hbm, t_hbm, o_hbm, p_v, t_v, acc_v, sem):
        wid = lax.axis_index("s") * NC + lax.axis_index("c")
        base = wid * ROWS_PER_W
        acc_v[0] = jnp.zeros((L,), jnp.float32)
        row_iota = lax.iota(jnp.int32, L)

        @pl.loop(0, N_CHUNKS)
        def _(ch):
            r0_ = base + ch * CHUNK
            cp1 = pltpu.async_copy(p_hbm.at[pl.ds(r0_, CHUNK)], p_v, sem)
            cp2 = pltpu.async_copy(t_hbm.at[pl.ds(r0_, CHUNK)], t_v, sem)
            cp1.wait()
            cp2.wait()

            @pl.loop(0, S)
            def _(j1):
                @pl.loop(0, S)
                def _(j2):
                    cbase = (j1 * S + j2) * N_ELEM
                    pg = [
                        plsc.load_gather(p_v, [row_iota, _splat(cbase + k)])
                        for k in range(10)
                    ]
                    tg = [
                        plsc.load_gather(t_v, [row_iota, _splat(cbase + k)])
                        for k in range(10)
                    ]
                    box = _box_loss(pg, tg)
                    coordf = jnp.where(tg[5] > 0.0, 1.0, 0.0)
                    cls = None
                    for k in range(10, 30):
                        pk = plsc.load_gather(p_v, [row_iota, _splat(cbase + k)])
                        tk = plsc.load_gather(t_v, [row_iota, _splat(cbase + k)])
                        dd = pk - tk
                        dd = dd * dd
                        cls = dd if cls is None else cls + dd
                    acc_v[0] = acc_v[0] + box + coordf * cls

        pltpu.sync_copy(acc_v, o_hbm.at[pl.ds(wid, 1)])

    return sc_kernel(pred2, targ2)


def _rl(x, k):
    return pltpu.roll(x, WP - k, 1)


def _tc_block_body(p_ref, t_ref, o_ref):
    zpad = jnp.zeros((TC_BLK, WP - W), jnp.float32)
    x = jnp.concatenate([p_ref[...], zpad], axis=1)
    y = jnp.concatenate([t_ref[...], zpad], axis=1)

    d = x - y
    d2 = d * d
    s1 = d2 + _rl(d2, 1)
    s2 = s1 + _rl(s1, 2)
    s4 = s2 + _rl(s2, 4)
    s8 = s4 + _rl(s4, 8)
    s_box0 = s2
    s_box1 = _rl(s2, 5)
    s_class = _rl(s8, 10) + _rl(s2, 26)
    c0 = _rl(d2, 4)
    c1 = _rl(d2, 9)

    x2, x5, x7 = _rl(x, 2), _rl(x, 5), _rl(x, 7)
    y2, y5, y7 = _rl(y, 2), _rl(y, 5), _rl(y, 7)

    def wh(ahi, bhi, alo, blo):
        return jnp.maximum(jnp.minimum(ahi, bhi) - jnp.maximum(alo, blo), 0.0)

    wh_c = wh(x2, y2, x, y)
    wh_a = wh(x7, y2, x5, y)
    wh_b = wh(x2, y7, x, y5)
    i_c = wh_c * _rl(wh_c, 1)
    i_a = wh_a * _rl(wh_a, 1)
    i_b = wh_b * _rl(wh_b, 1)

    ex = x2 - x
    ap = ex * _rl(ex, 1)
    ey = y2 - y
    at = ey * _rl(ey, 1)
    ap5 = _rl(ap, 5)
    at5 = _rl(at, 5)

    d_c = ap + at - i_c
    d_a = ap5 + at - i_a
    d_b = ap + at5 - i_b
    i_c5 = _rl(i_c, 5)
    d_c5 = _rl(d_c, 5)

    dl0 = i_a * d_c - i_c * d_a
    dl1 = i_c5 * d_b - i_b * d_c5
    m0 = jnp.where(d_a * d_c < 0.0, -dl0, dl0) > 0.0
    m1 = jnp.where(d_c5 * d_b < 0.0, -dl1, dl1) > 0.0

    coordf = jnp.where(y5 > 0.0, 1.0, 0.0)
    nw = jnp.where(y5 == 0.0, LAMBDA_NOOBJ, 0.0)
    r0 = jnp.where(~(m0 & m1), coordf, 0.0)
    r1 = jnp.where(m0 | m1, coordf, 0.0)

    per_cell = (
        LAMBDA_COORD * (r0 * s_box0 + r1 * s_box1)
        + r0 * c0 + r1 * c1
        + nw * (c0 + c1)
        + coordf * s_class
    )
    lane = jax.lax.broadcasted_iota(jnp.int32, (TC_BLK, WP), 1)
    masked = jnp.where((lane % N_ELEM == 0) & (lane < W), per_cell, 0.0)
    o_ref[...] = jnp.sum(masked).reshape(1, 1, 1)


def _tc_partials(p4, t4):
    return pl.pallas_call(
        _tc_block_body,
        grid=(TC_GRID,),
        in_specs=[
            pl.BlockSpec((TC_BLK, W), lambda i: (i, 0)),
            pl.BlockSpec((TC_BLK, W), lambda i: (i, 0)),
        ],
        out_specs=pl.BlockSpec((1, 1, 1), lambda i: (i, 0, 0)),
        out_shape=jax.ShapeDtypeStruct((TC_GRID, 1, 1), jnp.float32),
    )(p4, t4)


def kernel(pred_tensor, target_tensor):
    targ2 = target_tensor.reshape(BATCH, FEAT)
    tc_part = _tc_partials(
        pred_tensor[SC_ROWS:].reshape(N4, W),
        targ2[SC_ROWS:].reshape(N4, W),
    )
    sc_part = _sc_partials(pred_tensor[:SC_ROWS], targ2[:SC_ROWS])
    return jnp.sum(sc_part) + jnp.sum(tc_part)
